# baseline (speedup 1.0000x reference)
import jax
import jax.numpy as jnp
from jax.experimental import pallas as pl
from jax.experimental.pallas import tpu as pltpu

_BLK = 8000


def _body(src_hbm, e_hbm, tgt_hbm, out_hbm,
          src_b, e_b, tgt_b, out_b, in_sems, out_sems):
    i = pl.program_id(0)
    n_i = pl.num_programs(0)
    slot = jax.lax.rem(i, 2)
    nxt = jax.lax.rem(i + 1, 2)

    def start_in(step, buf_slot):
        off = step * _BLK
        pltpu.make_async_copy(
            src_hbm.at[pl.ds(off, _BLK)], src_b.at[buf_slot],
            in_sems.at[0, buf_slot]).start()
        pltpu.make_async_copy(
            tgt_hbm.at[pl.ds(off, _BLK)], tgt_b.at[buf_slot],
            in_sems.at[1, buf_slot]).start()
        pltpu.make_async_copy(
            e_hbm.at[pl.ds(off, _BLK)], e_b.at[buf_slot],
            in_sems.at[2, buf_slot]).start()

    @pl.when(i == 0)
    def _():
        start_in(0, 0)

    @pl.when(i + 1 < n_i)
    def _():
        start_in(i + 1, nxt)

    off = i * _BLK
    pltpu.make_async_copy(
        src_hbm.at[pl.ds(off, _BLK)], src_b.at[slot], in_sems.at[0, slot]).wait()
    pltpu.make_async_copy(
        tgt_hbm.at[pl.ds(off, _BLK)], tgt_b.at[slot], in_sems.at[1, slot]).wait()
    pltpu.make_async_copy(
        e_hbm.at[pl.ds(off, _BLK)], e_b.at[slot], in_sems.at[2, slot]).wait()

    @pl.when(i >= 2)
    def _():
        pltpu.make_async_copy(
            out_b.at[slot], out_hbm.at[pl.ds((i - 2) * _BLK, _BLK)],
            out_sems.at[slot]).wait()

    gate = jax.nn.sigmoid(jnp.mean(e_b[slot], axis=-1, keepdims=True))
    out_b[slot] = src_b[slot] * gate + tgt_b[slot]

    pltpu.make_async_copy(
        out_b.at[slot], out_hbm.at[pl.ds(off, _BLK)], out_sems.at[slot]).start()

    @pl.when(i == n_i - 1)
    def _():
        @pl.when(n_i >= 2)
        def _():
            pltpu.make_async_copy(
                out_b.at[nxt], out_hbm.at[pl.ds((i - 1) * _BLK, _BLK)],
                out_sems.at[nxt]).wait()
        pltpu.make_async_copy(
            out_b.at[slot], out_hbm.at[pl.ds(off, _BLK)],
            out_sems.at[slot]).wait()


def kernel(src, e, tgt):
    n, d = src.shape
    de = e.shape[1]
    return pl.pallas_call(
        _body,
        grid=(n // _BLK,),
        in_specs=[
            pl.BlockSpec(memory_space=pl.ANY),
            pl.BlockSpec(memory_space=pl.ANY),
            pl.BlockSpec(memory_space=pl.ANY),
        ],
        out_specs=pl.BlockSpec(memory_space=pl.ANY),
        out_shape=jax.ShapeDtypeStruct((n, d), src.dtype),
        scratch_shapes=[
            pltpu.VMEM((2, _BLK, d), src.dtype),
            pltpu.VMEM((2, _BLK, de), e.dtype),
            pltpu.VMEM((2, _BLK, d), tgt.dtype),
            pltpu.VMEM((2, _BLK, d), src.dtype),
            pltpu.SemaphoreType.DMA((3, 2)),
            pltpu.SemaphoreType.DMA((2,)),
        ],
        compiler_params=pltpu.CompilerParams(
            dimension_semantics=("arbitrary",),
        ),
    )(src, e, tgt)
